# Initial kernel scaffold; baseline (speedup 1.0000x reference)
import jax
import jax.numpy as jnp
from jax.experimental import pallas as pl


def kernel(token_ids, token_type_ids, word_table, token_type_table, position_table, ln_gamma, ln_beta):
    raise NotImplementedError("write your pallas kernel here")


# speedup vs baseline: 2.9849x; 2.9849x over previous
import functools

import jax
import jax.numpy as jnp
from jax import lax
from jax.experimental import pallas as pl
from jax.experimental.pallas import tpu as pltpu
from jax.experimental.pallas import tpu_sc as plsc

_EPS = 1e-12


def _sc_gather(ids_flat, word_table):
    n = ids_flat.shape[0]
    d = word_table.shape[1]
    nw = 32
    b_per_w = n // nw
    ch = 512
    n_ch = b_per_w // ch

    mesh = plsc.VectorSubcoreMesh(core_axis_name="c", subcore_axis_name="s")

    @functools.partial(
        pl.kernel,
        mesh=mesh,
        out_type=jax.ShapeDtypeStruct((n, d), jnp.float32),
        scratch_types=[
            pltpu.VMEM((ch,), jnp.int32),
            pltpu.VMEM((ch, d), jnp.float32),
            pltpu.SemaphoreType.DMA,
        ],
    )
    def gather_kernel(ids_hbm, table_hbm, out_hbm, idx_v, rows_v, sem):
        wid = lax.axis_index("s") * 2 + lax.axis_index("c")
        base = wid * b_per_w

        def body(i, carry):
            off = base + i * ch
            pltpu.sync_copy(ids_hbm.at[pl.ds(off, ch)], idx_v)
            pltpu.async_copy(table_hbm.at[idx_v], rows_v, sem).wait()
            pltpu.sync_copy(rows_v, out_hbm.at[pl.ds(off, ch)])
            return carry

        lax.fori_loop(0, n_ch, body, 0)

    return gather_kernel(ids_flat, word_table)


def _tc_add_ln(gathered, token_type_ids, token_type_table, position_table,
               ln_gamma, ln_beta):
    b, s = token_type_ids.shape
    d = position_table.shape[1]
    ttf = token_type_ids.astype(jnp.float32).reshape(b, s, 1)

    def body(g_ref, tt_ref, ttab_ref, pos_ref, gamma_ref, beta_ref, out_ref):
        x = g_ref[0]
        ttc = tt_ref[0]
        ttab = ttab_ref[...]
        t0 = ttab[0:1, :]
        t1 = ttab[1:2, :]
        sel = t0 + ttc * (t1 - t0)
        e = x + sel + pos_ref[...]
        mean = jnp.mean(e, axis=-1, keepdims=True)
        c = e - mean
        var = jnp.mean(c * c, axis=-1, keepdims=True)
        y = c * lax.rsqrt(var + _EPS)
        out_ref[0] = y * gamma_ref[...] + beta_ref[...]

    return pl.pallas_call(
        body,
        grid=(b,),
        in_specs=[
            pl.BlockSpec((1, s, d), lambda i: (i, 0, 0)),
            pl.BlockSpec((1, s, 1), lambda i: (i, 0, 0)),
            pl.BlockSpec((2, d), lambda i: (0, 0)),
            pl.BlockSpec((s, d), lambda i: (0, 0)),
            pl.BlockSpec((1, d), lambda i: (0, 0)),
            pl.BlockSpec((1, d), lambda i: (0, 0)),
        ],
        out_specs=pl.BlockSpec((1, s, d), lambda i: (i, 0, 0)),
        out_shape=jax.ShapeDtypeStruct((b, s, d), jnp.float32),
    )(gathered.reshape(b, s, d), ttf, token_type_table, position_table,
      ln_gamma.reshape(1, d), ln_beta.reshape(1, d))


def kernel(token_ids, token_type_ids, word_table, token_type_table,
           position_table, ln_gamma, ln_beta):
    b, s = token_ids.shape
    ids_flat = token_ids.reshape(b * s).astype(jnp.int32)
    gathered = _sc_gather(ids_flat, word_table)
    return _tc_add_ln(gathered, token_type_ids.astype(jnp.int32),
                      token_type_table, position_table, ln_gamma, ln_beta)
